# baseline (speedup 1.0000x reference)
import functools

import jax
import jax.numpy as jnp
from jax import lax
from jax.experimental import pallas as pl
from jax.experimental.pallas import tpu as pltpu
from jax.experimental.pallas import tpu_sc as plsc

_E = 8
_LANES = 16
_NC, _NS = 2, 16


def _f16_bits_to_f32(xi32):
    mag = jnp.left_shift(jnp.bitwise_and(xi32, 0x7FFF), 13)
    f = lax.bitcast_convert_type(mag, jnp.float32)
    f = f * jnp.float32(5.192296858534828e33)
    sign = jnp.left_shift(jnp.bitwise_and(xi32, jnp.int32(0x8000)), 16)
    return lax.bitcast_convert_type(
        jnp.bitwise_or(lax.bitcast_convert_type(f, jnp.int32), sign),
        jnp.float32)


def _round_f32_to_f16_precision(f):
    b = lax.bitcast_convert_type(f, jnp.int32)
    lsb = jnp.bitwise_and(jnp.right_shift(b, 13), 1)
    b = b + 0x0FFF + lsb
    b = jnp.bitwise_and(b, jnp.int32(-0x2000))
    return lax.bitcast_convert_type(b, jnp.float32)


def _gate_logits_body(w_ref, x_ref, out_ref):
    f = _f16_bits_to_f32(x_ref[...].astype(jnp.int32)).astype(jnp.bfloat16)
    logits = lax.dot_general(
        w_ref[...], f,
        dimension_numbers=(((1,), (1,)), ((), ())),
        preferred_element_type=jnp.float32,
    )
    out_ref[...] = _round_f32_to_f16_precision(logits[:_E])


def _gate_logits(x_i16, W16, block_rows):
    n_tokens, hidden = x_i16.shape
    grid = (n_tokens // block_rows,)
    return pl.pallas_call(
        _gate_logits_body,
        grid=grid,
        in_specs=[
            pl.BlockSpec((16, hidden), lambda i: (0, 0)),
            pl.BlockSpec((block_rows, hidden), lambda i: (i, 0)),
        ],
        out_specs=pl.BlockSpec((_E, block_rows), lambda i: (0, i)),
        out_shape=jax.ShapeDtypeStruct((_E, n_tokens), jnp.float32),
    )(W16, x_i16)


def _route_sc(logits_t):
    n_tokens = logits_t.shape[1]
    n_workers = _NC * _NS
    tpw = n_tokens // n_workers
    mesh = plsc.VectorSubcoreMesh(
        core_axis_name="c", subcore_axis_name="s",
        num_cores=_NC, num_subcores=_NS,
    )

    @functools.partial(
        pl.kernel,
        out_type=[
            jax.ShapeDtypeStruct((n_tokens,), jnp.float32),
            jax.ShapeDtypeStruct((n_tokens,), jnp.float32),
            jax.ShapeDtypeStruct((n_tokens,), jnp.int32),
            jax.ShapeDtypeStruct((n_tokens,), jnp.int32),
        ],
        mesh=mesh,
        scratch_types=[
            pltpu.VMEM((_E, tpw), jnp.float32),
            pltpu.VMEM((tpw,), jnp.float32),
            pltpu.VMEM((tpw,), jnp.float32),
            pltpu.VMEM((tpw,), jnp.int32),
            pltpu.VMEM((tpw,), jnp.int32),
        ],
    )
    def route(lt_hbm, w1_hbm, w2_hbm, i1_hbm, i2_hbm,
              lt_v, w1_v, w2_v, i1_v, i2_v):
        wid = lax.axis_index("s") * _NC + lax.axis_index("c")
        base = wid * tpw
        pltpu.sync_copy(lt_hbm.at[:, pl.ds(base, tpw)], lt_v)

        def body(g, _):
            off = g * _LANES
            neg_inf = jnp.full((_LANES,), -jnp.inf, jnp.float32)
            zero_i = jnp.zeros((_LANES,), jnp.int32)
            m1, i1 = neg_inf, zero_i
            m2, i2 = neg_inf, zero_i
            for e in range(_E):
                le = lt_v[e, pl.ds(off, _LANES)]
                gt1 = le > m1
                lo = jnp.minimum(le, m1)
                lo_i = jnp.where(gt1, i1, e)
                m1 = jnp.maximum(le, m1)
                i1 = jnp.where(gt1, e, i1)
                gt2 = lo > m2
                m2 = jnp.where(gt2, lo, m2)
                i2 = jnp.where(gt2, lo_i, i2)
            ed = jnp.exp(m2 - m1)
            s = 1.0 / (1.0 + ed)
            w1_v[pl.ds(off, _LANES)] = s
            w2_v[pl.ds(off, _LANES)] = ed * s
            i1_v[pl.ds(off, _LANES)] = i1
            i2_v[pl.ds(off, _LANES)] = i2
            return 0

        lax.fori_loop(0, tpw // _LANES, body, 0)
        pltpu.sync_copy(w1_v, w1_hbm.at[pl.ds(base, tpw)])
        pltpu.sync_copy(w2_v, w2_hbm.at[pl.ds(base, tpw)])
        pltpu.sync_copy(i1_v, i1_hbm.at[pl.ds(base, tpw)])
        pltpu.sync_copy(i2_v, i2_hbm.at[pl.ds(base, tpw)])

    return route(logits_t)


def kernel(x, W):
    n_tokens = x.shape[0]
    block_rows = 2048 if n_tokens % 2048 == 0 else n_tokens
    x_i16 = lax.bitcast_convert_type(x, jnp.int16)
    W16 = jnp.pad(W.astype(jnp.bfloat16), ((0, 16 - _E), (0, 0)))
    logits_t = _gate_logits(x_i16, W16, block_rows)
    w1, w2, i1, i2 = _route_sc(logits_t)
    topk_weights = jnp.stack([w1, w2], axis=-1).astype(x.dtype)
    topk_indices = jnp.stack([i1, i2], axis=-1)
    return (topk_weights, topk_indices)
